# baseline (speedup 1.0000x reference)
import functools

import jax
import jax.numpy as jnp
import numpy as np
from jax import lax
from jax.experimental import pallas as pl
from jax.experimental.pallas import tpu as pltpu
from jax.experimental.pallas import tpu_sc as plsc

CH = 128
NH = 8
NHID = 64
FD = CH + 2 * NHID
AVG_LOG = float(np.log(17.0))

NC, NS, NW = 2, 16, 32
RNG = 320
RPW = 5
NPAD = NW * RPW * RNG
BIG = 3.0e38

_PREC = jax.lax.Precision.HIGHEST


def _dot(a, b):
    return jnp.dot(a, b, preferred_element_type=jnp.float32, precision=_PREC)


def _ln_in(x, g, b, eps=1e-5):
    m = x.mean(-1, keepdims=True)
    v = ((x - m) ** 2).mean(-1, keepdims=True)
    return (x - m) / jnp.sqrt(v + eps) * g + b


def _transformer_kernel(x_ref, inw_ref, inb_ref, outw_ref, outb_ref,
                        ln1g_ref, ln1b_ref, ff1w_ref, ff1b_ref,
                        ff2w_ref, ff2b_ref, ln2g_ref, ln2b_ref,
                        tabng_ref, tabnb_ref, o_ref):
    BB = x_ref.shape[0]
    S = 16
    dh = CH // NH
    x = x_ref[...].reshape(BB * S, CH)
    qkv = _dot(x, inw_ref[...]) + inb_ref[...]
    q = qkv[:, :CH]
    k = qkv[:, CH:2 * CH]
    v = qkv[:, 2 * CH:]
    G = 256 // S
    n_grp = BB // G
    rows = G * S
    row_ids = jax.lax.broadcasted_iota(jnp.int32, (rows, rows), 0) // S
    col_ids = jax.lax.broadcasted_iota(jnp.int32, (rows, rows), 1) // S
    neg = jnp.float32(-1e30)
    mask = jnp.where(row_ids == col_ids, 0.0, neg)
    grp_outs = []
    for g in range(n_grp):
        sl = slice(g * rows, (g + 1) * rows)
        head_outs = []
        for h in range(NH):
            hs = slice(h * dh, (h + 1) * dh)
            qh = q[sl, hs]
            kh = k[sl, hs]
            vh = v[sl, hs]
            s = _dot(qh, kh.T) * (1.0 / np.sqrt(dh)) + mask
            s = s - jnp.max(s, axis=-1, keepdims=True)
            e = jnp.exp(s)
            a = e / jnp.sum(e, axis=-1, keepdims=True)
            head_outs.append(_dot(a, vh))
        grp_outs.append(jnp.concatenate(head_outs, axis=-1))
    o = jnp.concatenate(grp_outs, axis=0)
    att = _dot(o, outw_ref[...]) + outb_ref[...]
    h1 = _ln_in(x + att, ln1g_ref[...], ln1b_ref[...])
    ff = jnp.maximum(_dot(h1, ff1w_ref[...]) + ff1b_ref[...], 0.0)
    ff = _dot(ff, ff2w_ref[...]) + ff2b_ref[...]
    h2 = _ln_in(h1 + ff, ln2g_ref[...], ln2b_ref[...])
    h3 = _ln_in(h2, tabng_ref[...], tabnb_ref[...])
    o_ref[...] = h3.reshape(BB, S, CH)


def _run_transformer(x_tab, p, blk=128):
    B = x_tab.shape[0]
    vec = lambda a: a.reshape(1, -1)
    args = [
        x_tab,
        p['in_w'].T, vec(p['in_b']),
        p['out_w'].T, vec(p['out_b']),
        vec(p['ln1_g']), vec(p['ln1_b']),
        p['ff1_w'].T, vec(p['ff1_b']),
        p['ff2_w'].T, vec(p['ff2_b']),
        vec(p['ln2_g']), vec(p['ln2_b']),
        vec(p['tabn_g']), vec(p['tabn_b']),
    ]
    in_specs = [pl.BlockSpec((blk, 16, CH), lambda i: (i, 0, 0))]
    for a in args[1:]:
        sh = a.shape
        in_specs.append(pl.BlockSpec(sh, lambda i: tuple(0 for _ in sh)))
    return pl.pallas_call(
        _transformer_kernel,
        grid=(B // blk,),
        in_specs=in_specs,
        out_specs=pl.BlockSpec((blk, 16, CH), lambda i: (i, 0, 0)),
        out_shape=jax.ShapeDtypeStruct((B, 16, CH), jnp.float32),
    )(*args)


def _edge1_kernel(ea_ref, wt_ref, bt_ref, wr_ref, br_ref, t_ref, r_ref):
    ea = ea_ref[...]
    t_ref[...] = _dot(ea, wt_ref[...]) + bt_ref[...]
    r_ref[...] = _dot(ea, wr_ref[...]) + br_ref[...]


def _run_edge1(edge_attr, wt, bt, wr, br, blk=6400):
    E = edge_attr.shape[0]
    return pl.pallas_call(
        _edge1_kernel,
        grid=(E // blk,),
        in_specs=[
            pl.BlockSpec((blk, NHID), lambda i: (i, 0)),
            pl.BlockSpec((NHID, NHID), lambda i: (0, 0)),
            pl.BlockSpec((1, NHID), lambda i: (0, 0)),
            pl.BlockSpec((NHID, NHID), lambda i: (0, 0)),
            pl.BlockSpec((1, NHID), lambda i: (0, 0)),
        ],
        out_specs=[
            pl.BlockSpec((blk, NHID), lambda i: (i, 0)),
            pl.BlockSpec((blk, NHID), lambda i: (i, 0)),
        ],
        out_shape=[
            jax.ShapeDtypeStruct((E, NHID), jnp.float32),
            jax.ShapeDtypeStruct((E, NHID), jnp.float32),
        ],
    )(edge_attr, wt, bt.reshape(1, -1), wr, br.reshape(1, -1))


def _node_proj_kernel(x_ref, wd_ref, ws_ref, y_ref, z_ref):
    x = x_ref[...]
    y_ref[...] = _dot(x, wd_ref[...])
    z_ref[...] = _dot(x, ws_ref[...])


def _run_node_proj(x_pad, wd, ws, blk=6400):
    Np = x_pad.shape[0]
    return pl.pallas_call(
        _node_proj_kernel,
        grid=(Np // blk,),
        in_specs=[
            pl.BlockSpec((blk, NHID), lambda i: (i, 0)),
            pl.BlockSpec((NHID, NHID), lambda i: (0, 0)),
            pl.BlockSpec((NHID, NHID), lambda i: (0, 0)),
        ],
        out_specs=[
            pl.BlockSpec((blk, NHID), lambda i: (i, 0)),
            pl.BlockSpec((blk, NHID), lambda i: (i, 0)),
        ],
        out_shape=[
            jax.ShapeDtypeStruct((Np, NHID), jnp.float32),
            jax.ShapeDtypeStruct((Np, NHID), jnp.float32),
        ],
    )(x_pad, wd, ws)


def _node_stage_kernel(x_ref, y_ref, su_ref, suu_ref, mxu_ref, mnu_ref,
                       cnt_ref, postw_ref, postb_ref, linw_ref, linb_ref,
                       bng_ref, bnb_ref, w1a_ref, w1b_ref,
                       xo_ref, p_ref, q_ref):
    x = x_ref[...]
    y = y_ref[...]
    cnt = cnt_ref[...]
    cc = jnp.maximum(cnt, 1.0)
    pos = cnt > 0
    su = su_ref[...]
    suu = suu_ref[...]
    eu_m = su / cc
    euu_m = suu / cc
    mean = jnp.where(pos, y + eu_m, 0.0)
    std = jnp.sqrt(jnp.maximum(euu_m - eu_m * eu_m, 0.0) + 1e-5)
    mx = jnp.where(pos, y + mxu_ref[...], 0.0)
    mn = jnp.where(pos, y + mnu_ref[...], 0.0)
    agg = jnp.concatenate([mean, mx, mn, std], axis=-1)
    ld = jnp.log(cc + 1.0)
    out = jnp.concatenate(
        [agg, agg * (ld / AVG_LOG), agg * (AVG_LOG / ld)], axis=-1)
    full = jnp.concatenate([x, out], axis=-1)
    conv = _dot(full, postw_ref[...]) + postb_ref[...]
    conv = _dot(conv, linw_ref[...]) + linb_ref[...]
    bn = conv * bng_ref[...] + bnb_ref[...]
    xn = (x + jnp.maximum(bn, 0.0)) * 0.5
    xo_ref[...] = xn
    p_ref[...] = _dot(xn, w1a_ref[...])
    q_ref[...] = _dot(xn, w1b_ref[...])


def _run_node_stage(x_pad, y, su, suu, mxu, mnu, cnt, p, blk=3200):
    Np = x_pad.shape[0]
    bn_scale = (p['bn_g'] / np.sqrt(1.0 + 1e-5)).reshape(1, -1)
    args = [
        x_pad, y, su, suu, mxu, mnu, cnt.reshape(Np, 1),
        p['post_w'].T, p['post_b'].reshape(1, -1),
        p['lin_w'].T, p['lin_b'].reshape(1, -1),
        bn_scale, p['bn_b'].reshape(1, -1),
        p['eu1_w'][:, :NHID].T, p['eu1_w'][:, NHID:2 * NHID].T,
    ]
    in_specs = []
    for idx, a in enumerate(args):
        if idx < 7:
            in_specs.append(
                pl.BlockSpec((blk, a.shape[1]), lambda i: (i, 0)))
        else:
            sh = a.shape
            in_specs.append(pl.BlockSpec(sh, lambda i: (0, 0)))
    return pl.pallas_call(
        _node_stage_kernel,
        grid=(Np // blk,),
        in_specs=in_specs,
        out_specs=[
            pl.BlockSpec((blk, NHID), lambda i: (i, 0)),
            pl.BlockSpec((blk, NHID), lambda i: (i, 0)),
            pl.BlockSpec((blk, NHID), lambda i: (i, 0)),
        ],
        out_shape=[
            jax.ShapeDtypeStruct((Np, NHID), jnp.float32),
            jax.ShapeDtypeStruct((Np, NHID), jnp.float32),
            jax.ShapeDtypeStruct((Np, NHID), jnp.float32),
        ],
    )(*args)


def _edge2_kernel(ea_ref, pre_ref, w2_ref, b2_ref, o_ref):
    act = jnp.maximum(pre_ref[...], 0.0)
    eu = _dot(act, w2_ref[...]) + b2_ref[...]
    o_ref[...] = ea_ref[...] + eu * 0.5


def _run_edge2(edge_attr, eu_pre, w2, b2, blk=6400):
    E = edge_attr.shape[0]
    return pl.pallas_call(
        _edge2_kernel,
        grid=(E // blk,),
        in_specs=[
            pl.BlockSpec((blk, NHID), lambda i: (i, 0)),
            pl.BlockSpec((blk, NHID), lambda i: (i, 0)),
            pl.BlockSpec((NHID, NHID), lambda i: (0, 0)),
            pl.BlockSpec((1, NHID), lambda i: (0, 0)),
        ],
        out_specs=pl.BlockSpec((blk, NHID), lambda i: (i, 0)),
        out_shape=jax.ShapeDtypeStruct((E, NHID), jnp.float32),
    )(edge_attr, eu_pre, w2, b2.reshape(1, -1))


def _fused_kernel(x_ref, flig_ref, flib_ref, w1_ref, b1_ref, w2_ref, b2_ref,
                  w3_ref, b3_ref, fng_ref, fnb_ref, o_ref):
    x = x_ref[...]
    h = _ln_in(x, flig_ref[...], flib_ref[...])
    h = _dot(h, w1_ref[...]) + b1_ref[...]
    h = jnp.where(h > 0, h, 0.01 * h)
    h = _dot(h, w2_ref[...]) + b2_ref[...]
    h = jnp.where(h > 0, h, 0.01 * h)
    h = _dot(h, w3_ref[...]) + b3_ref[...]
    h = _ln_in(h, fng_ref[...], fnb_ref[...])
    o_ref[...] = (x + h) * 0.5


def _run_fused(x_in, p, blk=1024):
    B = x_in.shape[0]
    args = [
        x_in,
        p['fli_g'].reshape(1, -1), p['fli_b'].reshape(1, -1),
        p['f1_w'].T, p['f1_b'].reshape(1, -1),
        p['f2_w'].T, p['f2_b'].reshape(1, -1),
        p['f3_w'].T, p['f3_b'].reshape(1, -1),
        p['fn_g'].reshape(1, -1), p['fn_b'].reshape(1, -1),
    ]
    in_specs = [pl.BlockSpec((blk, FD), lambda i: (i, 0))]
    for a in args[1:]:
        sh = a.shape
        in_specs.append(pl.BlockSpec(sh, lambda i: (0, 0)))
    return pl.pallas_call(
        _fused_kernel,
        grid=(B // blk,),
        in_specs=in_specs,
        out_specs=pl.BlockSpec((blk, FD), lambda i: (i, 0)),
        out_shape=jax.ShapeDtypeStruct((B, FD), jnp.float32),
    )(*args)


_MESH = plsc.VectorSubcoreMesh(core_axis_name="c", subcore_axis_name="s")


def _wid():
    return lax.axis_index("s") * NC + lax.axis_index("c")


def _make_combine(E, ngather, C2=256):
    NCHUNK = E // C2
    assert NCHUNK * C2 == E
    HSUB = C2 // 128

    def body(*refs):
        base_hbm = refs[0]
        idx_hbms = refs[1:1 + ngather]
        tab_hbms = refs[1 + ngather:1 + 2 * ngather]
        out_hbm = refs[1 + 2 * ngather]
        rbuf = refs[2 + 2 * ngather]
        ibufs = refs[3 + 2 * ngather:3 + 2 * ngather + ngather]
        sema, semb, semc = refs[3 + 3 * ngather:6 + 3 * ngather]

        w = _wid()
        njw = (NCHUNK - 1 - w) // NW + 1

        def off_of(j):
            return (w + j * NW) * C2

        def ld_descs(j, r):
            off = off_of(j)
            ds = [pltpu.make_async_copy(
                base_hbm.at[pl.ds(off, C2)],
                rbuf.at[pl.ds(r * C2, C2)], sema.at[r])]
            for k in range(ngather):
                ds.append(pltpu.make_async_copy(
                    idx_hbms[k].at[pl.ds(off, C2)],
                    ibufs[k].at[pl.ds(r * C2, C2)], sema.at[r]))
            return ds

        def add_args(r):
            out = []
            for k in range(ngather):
                for h in range(HSUB):
                    out.append((
                        tab_hbms[k].at[ibufs[k].at[pl.ds(r * C2 + h * 128, 128)]],
                        rbuf.at[pl.ds(r * C2 + h * 128, 128)], semb.at[r]))
            return out

        def st_desc(j, r):
            return pltpu.make_async_copy(
                rbuf.at[pl.ds(r * C2, C2)],
                out_hbm.at[pl.ds(off_of(j), C2)], semc.at[r])

        def stage(i, r):
            j_ld = 3 * i + r
            j_add = j_ld - 1
            j_st = j_ld - 2
            r_add = (r - 1) % 3
            r_st = (r - 2) % 3

            @pl.when(jnp.logical_and(j_ld >= 3, j_ld - 3 < njw))
            def _():
                st_desc(j_ld - 3, r).wait()

            @pl.when(j_ld < njw)
            def _():
                for d in ld_descs(j_ld, r):
                    d.start()

            @pl.when(jnp.logical_and(j_add >= 0, j_add < njw))
            def _():
                for d in ld_descs(j_add, r_add):
                    d.wait()
                for (s, dd, sm) in add_args(r_add):
                    pltpu.async_copy(s, dd, sm, add=True)

            @pl.when(jnp.logical_and(j_st >= 0, j_st < njw))
            def _():
                for (s, dd, sm) in add_args(r_st):
                    pltpu.make_async_copy(s, dd, sm).wait()
                st_desc(j_st, r_st).start()

        def loop_body(i, _):
            for r in range(3):
                stage(i, r)
            return 0

        nloop = (njw + 5) // 3
        lax.fori_loop(0, nloop, loop_body, 0)

    sds = jax.ShapeDtypeStruct
    scratch = [
        pltpu.VMEM((3 * C2, NHID), jnp.float32),
    ]
    for _ in range(ngather):
        scratch.append(pltpu.VMEM((3 * C2,), jnp.int32))
    scratch += [pltpu.SemaphoreType.DMA((3,)),
                pltpu.SemaphoreType.DMA((3,)),
                pltpu.SemaphoreType.DMA((3,))]

    return functools.partial(
        pl.kernel, body,
        out_type=sds((E, NHID), jnp.float32),
        mesh=_MESH,
        scratch_types=scratch,
        compiler_params=pltpu.CompilerParams(use_tc_tiling_on_sc=False),
    )()


E_TOT = 800000
DW = 1600
NWIN = E_TOT // DW
TRIG = 1024
PB = 128
LCAP = TRIG + DW + 2 * PB + 64


def _pna_body(dst_hbm, u_hbm, su_hbm, suu_hbm, mx_hbm, mn_hbm, cnt_hbm,
              dwin, elist, dlist, ubuf,
              acc_su, acc_suu, acc_mx, acc_mn, acc_cnt,
              npend_ref, sem_w, sem_g):
    w = _wid()
    iota = lax.iota(jnp.int32, 16)
    zero16 = jnp.zeros((16,), jnp.float32)
    lane0 = iota == 0
    one16 = jnp.ones((16,), jnp.float32)

    def win_desc(wi, slot):
        return pltpu.make_async_copy(
            dst_hbm.at[pl.ds(wi * DW, DW)],
            dwin.at[pl.ds(slot * DW, DW)], sem_w.at[slot])

    def gat_desc(b, slot):
        return pltpu.make_async_copy(
            u_hbm.at[elist.at[pl.ds(b * PB, PB)]],
            ubuf.at[pl.ds(slot * PB, PB)], sem_g.at[slot])

    def rmw_batch(b, slot):
        def grp_body(g, _):
            locs = dlist[pl.ds(b * PB + 16 * g, 16)]
            for e in range(16):
                loc = locs[e]
                base = loc * NHID
                row = slot * PB + 16 * g + e
                for j in range(4):
                    sl = pl.ds(base + 16 * j, 16)
                    u = ubuf[row, pl.ds(16 * j, 16)]
                    acc_su[sl] = acc_su[sl] + u
                    acc_suu[sl] = acc_suu[sl] + u * u
                    acc_mx[sl] = jnp.maximum(acc_mx[sl], u)
                    acc_mn[sl] = jnp.minimum(acc_mn[sl], u)
                plsc.addupdate_scatter(
                    acc_cnt, [iota * 0 + loc], one16, mask=lane0)
            return 0
        lax.fori_loop(0, PB // 16, grp_body, 0)

    def process_pending():
        npend = npend_ref[0]

        @pl.when(npend > 0)
        def _():
            for k in range(2 * PB // 16):
                elist[pl.ds(npend + 16 * k, 16)] = jnp.zeros((16,), jnp.int32)
                dlist[pl.ds(npend + 16 * k, 16)] = jnp.full((16,), RNG,
                                                            jnp.int32)
            nb2 = (npend + 2 * PB - 1) // (2 * PB)
            nb_all = nb2 * 2
            gat_desc(0, 0).start()

            def pair_body(i, _):
                b0 = 2 * i
                b1 = b0 + 1

                @pl.when(b1 < nb_all)
                def _():
                    gat_desc(b1, 1).start()
                gat_desc(b0, 0).wait()
                rmw_batch(b0, 0)

                @pl.when(b1 < nb_all)
                def _():
                    @pl.when(b1 + 1 < nb_all)
                    def _():
                        gat_desc(b1 + 1, 0).start()
                    gat_desc(b1, 1).wait()
                    rmw_batch(b1, 1)
                return 0

            lax.fori_loop(0, nb2, pair_body, 0)
        npend_ref[0] = 0

    def filter_window(wbase, slot, lo):
        def filt(v, _):
            d = dwin[pl.ds(slot * DW + 16 * v, 16)]
            t = d - lo
            m = plsc.bitcast(t, jnp.uint32) < jnp.uint32(RNG)
            c = plsc.all_reduce_population_count(m)

            @pl.when(c[0] > 0)
            def _():
                mi = jnp.where(m, 1, 0).astype(jnp.int32)
                ci = plsc.cumsum(mi)
                npend = npend_ref[0]
                pos = (npend - 1) + ci
                eids = (wbase + 16 * v) + iota
                plsc.store_scatter(elist, [pos], eids, mask=m)
                plsc.store_scatter(dlist, [pos], t, mask=m)
                npend_ref[0] = npend + ci[15]
            return 0
        lax.fori_loop(0, DW // 16, filt, 0)

    def range_body(rr, _):
        r = w * RPW + rr
        lo = r * RNG

        def init_body(i, _):
            sl = pl.ds(16 * i, 16)
            acc_su[sl] = zero16
            acc_suu[sl] = zero16
            acc_mx[sl] = jnp.full((16,), -BIG, jnp.float32)
            acc_mn[sl] = jnp.full((16,), BIG, jnp.float32)
            return 0
        lax.fori_loop(0, (RNG + 1) * NHID // 16, init_body, 0)

        def initc_body(i, _):
            acc_cnt[pl.ds(16 * i, 16)] = zero16
            return 0
        lax.fori_loop(0, 352 // 16, initc_body, 0)

        npend_ref[0] = 0
        win_desc(0, 0).start()

        def win_pair(i, _):
            w0 = 2 * i
            win_desc(w0 + 1, 1).start()
            win_desc(w0, 0).wait()
            filter_window(w0 * DW, 0, lo)

            @pl.when(npend_ref[0] >= TRIG)
            def _():
                process_pending()

            @pl.when(w0 + 2 < NWIN)
            def _():
                win_desc(w0 + 2, 0).start()
            win_desc(w0 + 1, 1).wait()
            filter_window((w0 + 1) * DW, 1, lo)

            @pl.when(npend_ref[0] >= TRIG)
            def _():
                process_pending()
            return 0

        lax.fori_loop(0, NWIN // 2, win_pair, 0)
        process_pending()

        pltpu.sync_copy(acc_su.at[pl.ds(0, RNG * NHID)],
                        su_hbm.at[pl.ds(lo * NHID, RNG * NHID)])
        pltpu.sync_copy(acc_suu.at[pl.ds(0, RNG * NHID)],
                        suu_hbm.at[pl.ds(lo * NHID, RNG * NHID)])
        pltpu.sync_copy(acc_mx.at[pl.ds(0, RNG * NHID)],
                        mx_hbm.at[pl.ds(lo * NHID, RNG * NHID)])
        pltpu.sync_copy(acc_mn.at[pl.ds(0, RNG * NHID)],
                        mn_hbm.at[pl.ds(lo * NHID, RNG * NHID)])
        pltpu.sync_copy(acc_cnt.at[pl.ds(0, RNG)],
                        cnt_hbm.at[pl.ds(lo, RNG)])
        return 0

    lax.fori_loop(0, RPW, range_body, 0)


def _run_pna_sc(dst, u_mat):
    sds = jax.ShapeDtypeStruct
    out_type = [
        sds((NPAD * NHID,), jnp.float32),
        sds((NPAD * NHID,), jnp.float32),
        sds((NPAD * NHID,), jnp.float32),
        sds((NPAD * NHID,), jnp.float32),
        sds((NPAD,), jnp.float32),
    ]
    scratch = [
        pltpu.VMEM((2 * DW,), jnp.int32),
        pltpu.VMEM((LCAP,), jnp.int32),
        pltpu.VMEM((LCAP,), jnp.int32),
        pltpu.VMEM((2 * PB, NHID), jnp.float32),
        pltpu.VMEM(((RNG + 1) * NHID,), jnp.float32),
        pltpu.VMEM(((RNG + 1) * NHID,), jnp.float32),
        pltpu.VMEM(((RNG + 1) * NHID,), jnp.float32),
        pltpu.VMEM(((RNG + 1) * NHID,), jnp.float32),
        pltpu.VMEM((352,), jnp.float32),
        pltpu.SMEM((8,), jnp.int32),
        pltpu.SemaphoreType.DMA((2,)),
        pltpu.SemaphoreType.DMA((2,)),
    ]
    fn = functools.partial(
        pl.kernel, _pna_body, out_type=out_type, mesh=_MESH,
        scratch_types=scratch,
        compiler_params=pltpu.CompilerParams(use_tc_tiling_on_sc=False,
                                             needs_layout_passes=False))()
    return fn(dst, u_mat)


def kernel(x_tab, x_gnn, edge_attr, edge_index, target_edge_index, params):
    p = params
    N = x_gnn.shape[0]
    E = edge_attr.shape[0]
    B = x_tab.shape[0]
    src = edge_index[0]
    dst = edge_index[1]
    t0 = target_edge_index[0]
    t1 = target_edge_index[1]

    wd = p['pre_w'][:, :NHID].T
    ws = p['pre_w'][:, NHID:2 * NHID].T
    we = p['pre_w'][:, 2 * NHID:]
    wt = p['ee_w'].T @ we.T
    bt = p['ee_b'] @ we.T + p['pre_b']
    wr = p['eu1_w'][:, 2 * NHID:].T
    br = p['eu1_b']

    x_tab_ln = _run_transformer(x_tab, p)
    cls = x_tab_ln[:, 0, :]
    feat = x_tab_ln[:, 1:, :]

    t_mat, r_mat = _run_edge1(edge_attr, wt, bt, wr, br)

    x_pad = jnp.pad(x_gnn, ((0, NPAD - N), (0, 0)))
    y_pad, z_pad = _run_node_proj(x_pad, wd, ws)

    u_mat = _make_combine(E, 1)(t_mat, src, z_pad)
    su_f, suu_f, mx_f, mn_f, cnt = _run_pna_sc(dst, u_mat)
    su = su_f.reshape(NPAD, NHID)
    suu = suu_f.reshape(NPAD, NHID)
    mxu = mx_f.reshape(NPAD, NHID)
    mnu = mn_f.reshape(NPAD, NHID)

    xg_new, p_proj, q_proj = _run_node_stage(
        x_pad, y_pad, su, suu, mxu, mnu, cnt, p)

    eu_pre = _make_combine(E, 2)(r_mat, src, dst, p_proj, q_proj)
    edge_attr_new = _run_edge2(edge_attr, eu_pre,
                               p['eu2_w'].T, p['eu2_b'])

    xg0 = jnp.take(xg_new, t0, axis=0)
    xg1 = jnp.take(xg_new, t1, axis=0)
    x_in = jnp.concatenate([cls, xg0, xg1], axis=-1)
    x_out = _run_fused(x_in, p)
    cls_new = (cls + x_out[:, :CH]) * 0.5
    x_tab_out = jnp.concatenate([cls_new[:, None, :], feat], axis=1)

    index = jnp.concatenate([t0, t1], 0)
    emb = jnp.concatenate(
        [x_out[:, CH:CH + NHID], x_out[:, CH + NHID:]], 0)
    summed = jnp.zeros((N, NHID), jnp.float32).at[index].add(emb)
    cnts = jnp.zeros((N,), jnp.float32).at[index].add(1.0)
    pooled = summed / jnp.maximum(cnts, 1.0)[:, None]
    xg_n = xg_new[:N]
    x_gnn_out = jnp.where(cnts[:, None] > 0, (xg_n + pooled) * 0.5, xg_n)
    return (x_tab_out, x_gnn_out, edge_attr_new)


# speedup vs baseline: 1.3028x; 1.3028x over previous
import functools

import jax
import jax.numpy as jnp
import numpy as np
from jax import lax
from jax.experimental import pallas as pl
from jax.experimental.pallas import tpu as pltpu
from jax.experimental.pallas import tpu_sc as plsc

CH = 128
NH = 8
NHID = 64
FD = CH + 2 * NHID
AVG_LOG = float(np.log(17.0))

NC, NS, NW = 2, 16, 32
RNG = 320
RPW = 5
NPAD = NW * RPW * RNG
BIG = 3.0e38

_PREC = jax.lax.Precision.HIGHEST


def _dot(a, b):
    return jnp.dot(a, b, preferred_element_type=jnp.float32, precision=_PREC)


def _ln_in(x, g, b, eps=1e-5):
    m = x.mean(-1, keepdims=True)
    v = ((x - m) ** 2).mean(-1, keepdims=True)
    return (x - m) / jnp.sqrt(v + eps) * g + b


def _transformer_kernel(x_ref, inw_ref, inb_ref, outw_ref, outb_ref,
                        ln1g_ref, ln1b_ref, ff1w_ref, ff1b_ref,
                        ff2w_ref, ff2b_ref, ln2g_ref, ln2b_ref,
                        tabng_ref, tabnb_ref, o_ref):
    BB = x_ref.shape[0]
    S = 16
    dh = CH // NH
    x = x_ref[...].reshape(BB * S, CH)
    qkv = _dot(x, inw_ref[...]) + inb_ref[...]
    q = qkv[:, :CH]
    k = qkv[:, CH:2 * CH]
    v = qkv[:, 2 * CH:]
    G = 256 // S
    n_grp = BB // G
    rows = G * S
    row_ids = jax.lax.broadcasted_iota(jnp.int32, (rows, rows), 0) // S
    col_ids = jax.lax.broadcasted_iota(jnp.int32, (rows, rows), 1) // S
    neg = jnp.float32(-1e30)
    mask = jnp.where(row_ids == col_ids, 0.0, neg)
    grp_outs = []
    for g in range(n_grp):
        sl = slice(g * rows, (g + 1) * rows)
        head_outs = []
        for h in range(NH):
            hs = slice(h * dh, (h + 1) * dh)
            qh = q[sl, hs]
            kh = k[sl, hs]
            vh = v[sl, hs]
            s = _dot(qh, kh.T) * (1.0 / np.sqrt(dh)) + mask
            s = s - jnp.max(s, axis=-1, keepdims=True)
            e = jnp.exp(s)
            a = e / jnp.sum(e, axis=-1, keepdims=True)
            head_outs.append(_dot(a, vh))
        grp_outs.append(jnp.concatenate(head_outs, axis=-1))
    o = jnp.concatenate(grp_outs, axis=0)
    att = _dot(o, outw_ref[...]) + outb_ref[...]
    h1 = _ln_in(x + att, ln1g_ref[...], ln1b_ref[...])
    ff = jnp.maximum(_dot(h1, ff1w_ref[...]) + ff1b_ref[...], 0.0)
    ff = _dot(ff, ff2w_ref[...]) + ff2b_ref[...]
    h2 = _ln_in(h1 + ff, ln2g_ref[...], ln2b_ref[...])
    h3 = _ln_in(h2, tabng_ref[...], tabnb_ref[...])
    o_ref[...] = h3.reshape(BB, S, CH)


def _run_transformer(x_tab, p, blk=128):
    B = x_tab.shape[0]
    vec = lambda a: a.reshape(1, -1)
    args = [
        x_tab,
        p['in_w'].T, vec(p['in_b']),
        p['out_w'].T, vec(p['out_b']),
        vec(p['ln1_g']), vec(p['ln1_b']),
        p['ff1_w'].T, vec(p['ff1_b']),
        p['ff2_w'].T, vec(p['ff2_b']),
        vec(p['ln2_g']), vec(p['ln2_b']),
        vec(p['tabn_g']), vec(p['tabn_b']),
    ]
    in_specs = [pl.BlockSpec((blk, 16, CH), lambda i: (i, 0, 0))]
    for a in args[1:]:
        sh = a.shape
        in_specs.append(pl.BlockSpec(sh, lambda i: tuple(0 for _ in sh)))
    return pl.pallas_call(
        _transformer_kernel,
        grid=(B // blk,),
        in_specs=in_specs,
        out_specs=pl.BlockSpec((blk, 16, CH), lambda i: (i, 0, 0)),
        out_shape=jax.ShapeDtypeStruct((B, 16, CH), jnp.float32),
    )(*args)


def _edge1_kernel(ea_ref, wt_ref, bt_ref, wr_ref, br_ref, t_ref, r_ref):
    ea = ea_ref[...]
    t_ref[...] = _dot(ea, wt_ref[...]) + bt_ref[...]
    r_ref[...] = _dot(ea, wr_ref[...]) + br_ref[...]


def _run_edge1(edge_attr, wt, bt, wr, br, blk=6400):
    E = edge_attr.shape[0]
    return pl.pallas_call(
        _edge1_kernel,
        grid=(E // blk,),
        in_specs=[
            pl.BlockSpec((blk, NHID), lambda i: (i, 0)),
            pl.BlockSpec((NHID, NHID), lambda i: (0, 0)),
            pl.BlockSpec((1, NHID), lambda i: (0, 0)),
            pl.BlockSpec((NHID, NHID), lambda i: (0, 0)),
            pl.BlockSpec((1, NHID), lambda i: (0, 0)),
        ],
        out_specs=[
            pl.BlockSpec((blk, NHID), lambda i: (i, 0)),
            pl.BlockSpec((blk, NHID), lambda i: (i, 0)),
        ],
        out_shape=[
            jax.ShapeDtypeStruct((E, NHID), jnp.float32),
            jax.ShapeDtypeStruct((E, NHID), jnp.float32),
        ],
    )(edge_attr, wt, bt.reshape(1, -1), wr, br.reshape(1, -1))


def _node_proj_kernel(x_ref, wd_ref, ws_ref, y_ref, z_ref):
    x = x_ref[...]
    y_ref[...] = _dot(x, wd_ref[...])
    z_ref[...] = _dot(x, ws_ref[...])


def _run_node_proj(x_pad, wd, ws, blk=6400):
    Np = x_pad.shape[0]
    return pl.pallas_call(
        _node_proj_kernel,
        grid=(Np // blk,),
        in_specs=[
            pl.BlockSpec((blk, NHID), lambda i: (i, 0)),
            pl.BlockSpec((NHID, NHID), lambda i: (0, 0)),
            pl.BlockSpec((NHID, NHID), lambda i: (0, 0)),
        ],
        out_specs=[
            pl.BlockSpec((blk, NHID), lambda i: (i, 0)),
            pl.BlockSpec((blk, NHID), lambda i: (i, 0)),
        ],
        out_shape=[
            jax.ShapeDtypeStruct((Np, NHID), jnp.float32),
            jax.ShapeDtypeStruct((Np, NHID), jnp.float32),
        ],
    )(x_pad, wd, ws)


def _node_stage_kernel(x_ref, y_ref, su_ref, suu_ref, mxu_ref, mnu_ref,
                       cnt_ref, postw_ref, postb_ref, linw_ref, linb_ref,
                       bng_ref, bnb_ref, w1a_ref, w1b_ref,
                       xo_ref, p_ref, q_ref):
    x = x_ref[...]
    y = y_ref[...]
    cnt = cnt_ref[...]
    cc = jnp.maximum(cnt, 1.0)
    pos = cnt > 0
    su = su_ref[...]
    suu = suu_ref[...]
    eu_m = su / cc
    euu_m = suu / cc
    mean = jnp.where(pos, y + eu_m, 0.0)
    std = jnp.sqrt(jnp.maximum(euu_m - eu_m * eu_m, 0.0) + 1e-5)
    mx = jnp.where(pos, y + mxu_ref[...], 0.0)
    mn = jnp.where(pos, y + mnu_ref[...], 0.0)
    agg = jnp.concatenate([mean, mx, mn, std], axis=-1)
    ld = jnp.log(cc + 1.0)
    out = jnp.concatenate(
        [agg, agg * (ld / AVG_LOG), agg * (AVG_LOG / ld)], axis=-1)
    full = jnp.concatenate([x, out], axis=-1)
    conv = _dot(full, postw_ref[...]) + postb_ref[...]
    conv = _dot(conv, linw_ref[...]) + linb_ref[...]
    bn = conv * bng_ref[...] + bnb_ref[...]
    xn = (x + jnp.maximum(bn, 0.0)) * 0.5
    xo_ref[...] = xn
    p_ref[...] = _dot(xn, w1a_ref[...])
    q_ref[...] = _dot(xn, w1b_ref[...])


def _run_node_stage(x_pad, y, su, suu, mxu, mnu, cnt, p, blk=3200):
    Np = x_pad.shape[0]
    bn_scale = (p['bn_g'] / np.sqrt(1.0 + 1e-5)).reshape(1, -1)
    args = [
        x_pad, y, su, suu, mxu, mnu, cnt.reshape(Np, 1),
        p['post_w'].T, p['post_b'].reshape(1, -1),
        p['lin_w'].T, p['lin_b'].reshape(1, -1),
        bn_scale, p['bn_b'].reshape(1, -1),
        p['eu1_w'][:, :NHID].T, p['eu1_w'][:, NHID:2 * NHID].T,
    ]
    in_specs = []
    for idx, a in enumerate(args):
        if idx < 7:
            in_specs.append(
                pl.BlockSpec((blk, a.shape[1]), lambda i: (i, 0)))
        else:
            sh = a.shape
            in_specs.append(pl.BlockSpec(sh, lambda i: (0, 0)))
    return pl.pallas_call(
        _node_stage_kernel,
        grid=(Np // blk,),
        in_specs=in_specs,
        out_specs=[
            pl.BlockSpec((blk, NHID), lambda i: (i, 0)),
            pl.BlockSpec((blk, NHID), lambda i: (i, 0)),
            pl.BlockSpec((blk, NHID), lambda i: (i, 0)),
        ],
        out_shape=[
            jax.ShapeDtypeStruct((Np, NHID), jnp.float32),
            jax.ShapeDtypeStruct((Np, NHID), jnp.float32),
            jax.ShapeDtypeStruct((Np, NHID), jnp.float32),
        ],
    )(*args)


def _edge2_kernel(ea_ref, pre_ref, w2_ref, b2_ref, o_ref):
    act = jnp.maximum(pre_ref[...], 0.0)
    eu = _dot(act, w2_ref[...]) + b2_ref[...]
    o_ref[...] = ea_ref[...] + eu * 0.5


def _run_edge2(edge_attr, eu_pre, w2, b2, blk=6400):
    E = edge_attr.shape[0]
    return pl.pallas_call(
        _edge2_kernel,
        grid=(E // blk,),
        in_specs=[
            pl.BlockSpec((blk, NHID), lambda i: (i, 0)),
            pl.BlockSpec((blk, NHID), lambda i: (i, 0)),
            pl.BlockSpec((NHID, NHID), lambda i: (0, 0)),
            pl.BlockSpec((1, NHID), lambda i: (0, 0)),
        ],
        out_specs=pl.BlockSpec((blk, NHID), lambda i: (i, 0)),
        out_shape=jax.ShapeDtypeStruct((E, NHID), jnp.float32),
    )(edge_attr, eu_pre, w2, b2.reshape(1, -1))


def _fused_kernel(x_ref, flig_ref, flib_ref, w1_ref, b1_ref, w2_ref, b2_ref,
                  w3_ref, b3_ref, fng_ref, fnb_ref, o_ref):
    x = x_ref[...]
    h = _ln_in(x, flig_ref[...], flib_ref[...])
    h = _dot(h, w1_ref[...]) + b1_ref[...]
    h = jnp.where(h > 0, h, 0.01 * h)
    h = _dot(h, w2_ref[...]) + b2_ref[...]
    h = jnp.where(h > 0, h, 0.01 * h)
    h = _dot(h, w3_ref[...]) + b3_ref[...]
    h = _ln_in(h, fng_ref[...], fnb_ref[...])
    o_ref[...] = (x + h) * 0.5


def _run_fused(x_in, p, blk=1024):
    B = x_in.shape[0]
    args = [
        x_in,
        p['fli_g'].reshape(1, -1), p['fli_b'].reshape(1, -1),
        p['f1_w'].T, p['f1_b'].reshape(1, -1),
        p['f2_w'].T, p['f2_b'].reshape(1, -1),
        p['f3_w'].T, p['f3_b'].reshape(1, -1),
        p['fn_g'].reshape(1, -1), p['fn_b'].reshape(1, -1),
    ]
    in_specs = [pl.BlockSpec((blk, FD), lambda i: (i, 0))]
    for a in args[1:]:
        sh = a.shape
        in_specs.append(pl.BlockSpec(sh, lambda i: (0, 0)))
    return pl.pallas_call(
        _fused_kernel,
        grid=(B // blk,),
        in_specs=in_specs,
        out_specs=pl.BlockSpec((blk, FD), lambda i: (i, 0)),
        out_shape=jax.ShapeDtypeStruct((B, FD), jnp.float32),
    )(*args)


_MESH = plsc.VectorSubcoreMesh(core_axis_name="c", subcore_axis_name="s")


def _wid():
    return lax.axis_index("s") * NC + lax.axis_index("c")


def _make_combine(E, ngather, C2=256):
    NCHUNK = E // C2
    assert NCHUNK * C2 == E
    HSUB = C2 // 128

    def body(*refs):
        base_hbm = refs[0]
        idx_hbms = refs[1:1 + ngather]
        tab_hbms = refs[1 + ngather:1 + 2 * ngather]
        out_hbm = refs[1 + 2 * ngather]
        rbuf = refs[2 + 2 * ngather]
        ibufs = refs[3 + 2 * ngather:3 + 2 * ngather + ngather]
        sema, semb, semc = refs[3 + 3 * ngather:6 + 3 * ngather]

        w = _wid()
        njw = (NCHUNK - 1 - w) // NW + 1

        def off_of(j):
            return (w + j * NW) * C2

        def ld_descs(j, r):
            off = off_of(j)
            ds = [pltpu.make_async_copy(
                base_hbm.at[pl.ds(off, C2)],
                rbuf.at[pl.ds(r * C2, C2)], sema.at[r])]
            for k in range(ngather):
                ds.append(pltpu.make_async_copy(
                    idx_hbms[k].at[pl.ds(off, C2)],
                    ibufs[k].at[pl.ds(r * C2, C2)], sema.at[r]))
            return ds

        def add_args(r):
            out = []
            for k in range(ngather):
                for h in range(HSUB):
                    out.append((
                        tab_hbms[k].at[ibufs[k].at[pl.ds(r * C2 + h * 128, 128)]],
                        rbuf.at[pl.ds(r * C2 + h * 128, 128)], semb.at[r]))
            return out

        def st_desc(j, r):
            return pltpu.make_async_copy(
                rbuf.at[pl.ds(r * C2, C2)],
                out_hbm.at[pl.ds(off_of(j), C2)], semc.at[r])

        def stage(i, r):
            j_ld = 3 * i + r
            j_add = j_ld - 1
            j_st = j_ld - 2
            r_add = (r - 1) % 3
            r_st = (r - 2) % 3

            @pl.when(jnp.logical_and(j_ld >= 3, j_ld - 3 < njw))
            def _():
                st_desc(j_ld - 3, r).wait()

            @pl.when(j_ld < njw)
            def _():
                for d in ld_descs(j_ld, r):
                    d.start()

            @pl.when(jnp.logical_and(j_add >= 0, j_add < njw))
            def _():
                for d in ld_descs(j_add, r_add):
                    d.wait()
                for (s, dd, sm) in add_args(r_add):
                    pltpu.async_copy(s, dd, sm, add=True)

            @pl.when(jnp.logical_and(j_st >= 0, j_st < njw))
            def _():
                for (s, dd, sm) in add_args(r_st):
                    pltpu.make_async_copy(s, dd, sm).wait()
                st_desc(j_st, r_st).start()

        def loop_body(i, _):
            for r in range(3):
                stage(i, r)
            return 0

        nloop = (njw + 5) // 3
        lax.fori_loop(0, nloop, loop_body, 0)

    sds = jax.ShapeDtypeStruct
    scratch = [
        pltpu.VMEM((3 * C2, NHID), jnp.float32),
    ]
    for _ in range(ngather):
        scratch.append(pltpu.VMEM((3 * C2,), jnp.int32))
    scratch += [pltpu.SemaphoreType.DMA((3,)),
                pltpu.SemaphoreType.DMA((3,)),
                pltpu.SemaphoreType.DMA((3,))]

    return functools.partial(
        pl.kernel, body,
        out_type=sds((E, NHID), jnp.float32),
        mesh=_MESH,
        scratch_types=scratch,
        compiler_params=pltpu.CompilerParams(use_tc_tiling_on_sc=False),
    )()


E_TOT = 800000
DW = 1600
NWIN = E_TOT // DW
TRIG = 1024
PB = 128
LCAP = TRIG + DW + 2 * PB + 64


def _pna_body(dst_hbm, u_hbm, su_hbm, suu_hbm, mx_hbm, mn_hbm, cnt_hbm,
              dwin, elist, dlist, ubuf,
              acc_su, acc_suu, acc_mx, acc_mn, acc_cnt,
              npend_ref, sem_w, sem_g):
    w = _wid()
    iota = lax.iota(jnp.int32, 16)
    zero16 = jnp.zeros((16,), jnp.float32)
    lane0 = iota == 0
    one16 = jnp.ones((16,), jnp.float32)

    def win_desc(wi, slot):
        return pltpu.make_async_copy(
            dst_hbm.at[pl.ds(wi * DW, DW)],
            dwin.at[pl.ds(slot * DW, DW)], sem_w.at[slot])

    def gat_desc(b, slot):
        return pltpu.make_async_copy(
            u_hbm.at[elist.at[pl.ds(b * PB, PB)]],
            ubuf.at[pl.ds(slot * PB, PB)], sem_g.at[slot])

    def rmw_batch(b, slot):
        def grp_body(g, _):
            locs = dlist[pl.ds(b * PB + 16 * g, 16)]
            for e in range(16):
                loc = locs[e]
                base = loc * NHID
                row = slot * PB + 16 * g + e
                for j in range(4):
                    sl = pl.ds(base + 16 * j, 16)
                    u = ubuf[row, pl.ds(16 * j, 16)]
                    acc_su[sl] = acc_su[sl] + u
                    acc_suu[sl] = acc_suu[sl] + u * u
                    acc_mx[sl] = jnp.maximum(acc_mx[sl], u)
                    acc_mn[sl] = jnp.minimum(acc_mn[sl], u)
                plsc.addupdate_scatter(
                    acc_cnt, [iota * 0 + loc], one16, mask=lane0)
            return 0
        lax.fori_loop(0, PB // 16, grp_body, 0)

    def process_pending():
        npend = npend_ref[0]

        @pl.when(npend > 0)
        def _():
            for k in range(2 * PB // 16):
                elist[pl.ds(npend + 16 * k, 16)] = jnp.zeros((16,), jnp.int32)
                dlist[pl.ds(npend + 16 * k, 16)] = jnp.full((16,), RNG,
                                                            jnp.int32)
            nb2 = (npend + 2 * PB - 1) // (2 * PB)
            nb_all = nb2 * 2
            gat_desc(0, 0).start()

            def pair_body(i, _):
                b0 = 2 * i
                b1 = b0 + 1

                @pl.when(b1 < nb_all)
                def _():
                    gat_desc(b1, 1).start()
                gat_desc(b0, 0).wait()
                rmw_batch(b0, 0)

                @pl.when(b1 < nb_all)
                def _():
                    @pl.when(b1 + 1 < nb_all)
                    def _():
                        gat_desc(b1 + 1, 0).start()
                    gat_desc(b1, 1).wait()
                    rmw_batch(b1, 1)
                return 0

            lax.fori_loop(0, nb2, pair_body, 0)
        npend_ref[0] = 0

    def filter_window(wbase, slot, lo):
        def filt(v, _):
            d = dwin[pl.ds(slot * DW + 16 * v, 16)]
            t = d - lo
            m = jnp.logical_and(t >= 0, t < RNG)
            mi = jnp.where(m, 1, 0).astype(jnp.int32)
            ci = plsc.cumsum(mi)
            npend = npend_ref[0]
            pos = (npend - 1) + ci
            eids = (wbase + 16 * v) + iota
            plsc.store_scatter(elist, [pos], eids, mask=m)
            plsc.store_scatter(dlist, [pos], t, mask=m)
            npend_ref[0] = npend + ci[15]
            return 0
        lax.fori_loop(0, DW // 16, filt, 0)

    def range_body(rr, _):
        r = w * RPW + rr
        lo = r * RNG

        def init_body(i, _):
            sl = pl.ds(16 * i, 16)
            acc_su[sl] = zero16
            acc_suu[sl] = zero16
            acc_mx[sl] = jnp.full((16,), -BIG, jnp.float32)
            acc_mn[sl] = jnp.full((16,), BIG, jnp.float32)
            return 0
        lax.fori_loop(0, (RNG + 1) * NHID // 16, init_body, 0)

        def initc_body(i, _):
            acc_cnt[pl.ds(16 * i, 16)] = zero16
            return 0
        lax.fori_loop(0, 352 // 16, initc_body, 0)

        npend_ref[0] = 0
        win_desc(0, 0).start()

        def win_pair(i, _):
            w0 = 2 * i
            win_desc(w0 + 1, 1).start()
            win_desc(w0, 0).wait()
            filter_window(w0 * DW, 0, lo)

            @pl.when(npend_ref[0] >= TRIG)
            def _():
                process_pending()

            @pl.when(w0 + 2 < NWIN)
            def _():
                win_desc(w0 + 2, 0).start()
            win_desc(w0 + 1, 1).wait()
            filter_window((w0 + 1) * DW, 1, lo)

            @pl.when(npend_ref[0] >= TRIG)
            def _():
                process_pending()
            return 0

        lax.fori_loop(0, NWIN // 2, win_pair, 0)
        process_pending()

        pltpu.sync_copy(acc_su.at[pl.ds(0, RNG * NHID)],
                        su_hbm.at[pl.ds(lo * NHID, RNG * NHID)])
        pltpu.sync_copy(acc_suu.at[pl.ds(0, RNG * NHID)],
                        suu_hbm.at[pl.ds(lo * NHID, RNG * NHID)])
        pltpu.sync_copy(acc_mx.at[pl.ds(0, RNG * NHID)],
                        mx_hbm.at[pl.ds(lo * NHID, RNG * NHID)])
        pltpu.sync_copy(acc_mn.at[pl.ds(0, RNG * NHID)],
                        mn_hbm.at[pl.ds(lo * NHID, RNG * NHID)])
        pltpu.sync_copy(acc_cnt.at[pl.ds(0, RNG)],
                        cnt_hbm.at[pl.ds(lo, RNG)])
        return 0

    lax.fori_loop(0, RPW, range_body, 0)


def _run_pna_sc(dst, u_mat):
    sds = jax.ShapeDtypeStruct
    out_type = [
        sds((NPAD * NHID,), jnp.float32),
        sds((NPAD * NHID,), jnp.float32),
        sds((NPAD * NHID,), jnp.float32),
        sds((NPAD * NHID,), jnp.float32),
        sds((NPAD,), jnp.float32),
    ]
    scratch = [
        pltpu.VMEM((2 * DW,), jnp.int32),
        pltpu.VMEM((LCAP,), jnp.int32),
        pltpu.VMEM((LCAP,), jnp.int32),
        pltpu.VMEM((2 * PB, NHID), jnp.float32),
        pltpu.VMEM(((RNG + 1) * NHID,), jnp.float32),
        pltpu.VMEM(((RNG + 1) * NHID,), jnp.float32),
        pltpu.VMEM(((RNG + 1) * NHID,), jnp.float32),
        pltpu.VMEM(((RNG + 1) * NHID,), jnp.float32),
        pltpu.VMEM((352,), jnp.float32),
        pltpu.SMEM((8,), jnp.int32),
        pltpu.SemaphoreType.DMA((2,)),
        pltpu.SemaphoreType.DMA((2,)),
    ]
    fn = functools.partial(
        pl.kernel, _pna_body, out_type=out_type, mesh=_MESH,
        scratch_types=scratch,
        compiler_params=pltpu.CompilerParams(use_tc_tiling_on_sc=False,
                                             needs_layout_passes=False))()
    return fn(dst, u_mat)


def kernel(x_tab, x_gnn, edge_attr, edge_index, target_edge_index, params):
    p = params
    N = x_gnn.shape[0]
    E = edge_attr.shape[0]
    B = x_tab.shape[0]
    src = edge_index[0]
    dst = edge_index[1]
    t0 = target_edge_index[0]
    t1 = target_edge_index[1]

    wd = p['pre_w'][:, :NHID].T
    ws = p['pre_w'][:, NHID:2 * NHID].T
    we = p['pre_w'][:, 2 * NHID:]
    wt = p['ee_w'].T @ we.T
    bt = p['ee_b'] @ we.T + p['pre_b']
    wr = p['eu1_w'][:, 2 * NHID:].T
    br = p['eu1_b']

    x_tab_ln = _run_transformer(x_tab, p)
    cls = x_tab_ln[:, 0, :]
    feat = x_tab_ln[:, 1:, :]

    t_mat, r_mat = _run_edge1(edge_attr, wt, bt, wr, br)

    x_pad = jnp.pad(x_gnn, ((0, NPAD - N), (0, 0)))
    y_pad, z_pad = _run_node_proj(x_pad, wd, ws)

    u_mat = _make_combine(E, 1)(t_mat, src, z_pad)
    su_f, suu_f, mx_f, mn_f, cnt = _run_pna_sc(dst, u_mat)
    su = su_f.reshape(NPAD, NHID)
    suu = suu_f.reshape(NPAD, NHID)
    mxu = mx_f.reshape(NPAD, NHID)
    mnu = mn_f.reshape(NPAD, NHID)

    xg_new, p_proj, q_proj = _run_node_stage(
        x_pad, y_pad, su, suu, mxu, mnu, cnt, p)

    eu_pre = _make_combine(E, 2)(r_mat, src, dst, p_proj, q_proj)
    edge_attr_new = _run_edge2(edge_attr, eu_pre,
                               p['eu2_w'].T, p['eu2_b'])

    xg0 = jnp.take(xg_new, t0, axis=0)
    xg1 = jnp.take(xg_new, t1, axis=0)
    x_in = jnp.concatenate([cls, xg0, xg1], axis=-1)
    x_out = _run_fused(x_in, p)
    cls_new = (cls + x_out[:, :CH]) * 0.5
    x_tab_out = jnp.concatenate([cls_new[:, None, :], feat], axis=1)

    index = jnp.concatenate([t0, t1], 0)
    emb = jnp.concatenate(
        [x_out[:, CH:CH + NHID], x_out[:, CH + NHID:]], 0)
    summed = jnp.zeros((N, NHID), jnp.float32).at[index].add(emb)
    cnts = jnp.zeros((N,), jnp.float32).at[index].add(1.0)
    pooled = summed / jnp.maximum(cnts, 1.0)[:, None]
    xg_n = xg_new[:N]
    x_gnn_out = jnp.where(cnts[:, None] > 0, (xg_n + pooled) * 0.5, xg_n)
    return (x_tab_out, x_gnn_out, edge_attr_new)
